# baseline (speedup 1.0000x reference)
import functools

import jax
import jax.numpy as jnp
import numpy as np
from jax import lax
from jax.experimental import pallas as pl
from jax.experimental.pallas import tpu as pltpu
from jax.experimental.pallas import tpu_sc as plsc

NC = 2
NS = 16
NW = NC * NS

D = 64
PERIOD = 200
CHUNK = 100
VOCAB = 1000000
VMAIN = 999936
NCT = VMAIN // 128


def _pos_table(seq_len, d_model):
    pos = jnp.arange(seq_len, dtype=jnp.float32)[:, None]
    div = jnp.exp(jnp.arange(0, d_model, 2, dtype=jnp.float32)
                  * -(np.log(10000.0) / d_model))
    pe = jnp.zeros((seq_len, d_model), dtype=jnp.float32)
    pe = pe.at[:, 0::2].set(jnp.sin(pos * div))
    pe = pe.at[:, 1::2].set(jnp.cos(pos * div))
    return pe


def _phase_a():
    mesh = plsc.VectorSubcoreMesh(core_axis_name="c", subcore_axis_name="s")

    @functools.partial(
        pl.kernel,
        mesh=mesh,
        compiler_params=pltpu.CompilerParams(
            use_tc_tiling_on_sc=True, needs_layout_passes=False),
        out_type=jax.ShapeDtypeStruct((VOCAB // 2, 128), jnp.float32),
        scratch_types=[
            pltpu.VMEM((2, D, 128), jnp.float32),
            pltpu.VMEM((2, D, 128), jnp.float32),
            pltpu.VMEM((32, 128), jnp.float32),
            pltpu.SemaphoreType.DMA((2,)),
            pltpu.SemaphoreType.DMA((2,)),
        ],
    )
    def body(tabT_hbm, tailp_hbm, out_hbm, in_v, tr_v, tail_v, isem, osem):
        wid = lax.axis_index("s") * NC + lax.axis_index("c")
        nct = 244 + (wid < 4)
        ct0 = wid * 244 + jnp.minimum(wid, 4)

        lanes = jnp.arange(16, dtype=jnp.int32)

        def fire_in(i, b):
            pltpu.async_copy(
                tabT_hbm.at[:, pl.ds((ct0 + i) * 128, 128)], in_v.at[b],
                isem.at[b])

        def wait_in(i, b):
            pltpu.make_async_copy(
                tabT_hbm.at[:, pl.ds((ct0 + i) * 128, 128)], in_v.at[b],
                isem.at[b]).wait()

        def fire_out(i, b):
            pltpu.async_copy(
                tr_v.at[b], out_hbm.at[pl.ds((ct0 + i) * 64, 64)], osem.at[b])

        def wait_out(i, b):
            pltpu.make_async_copy(
                tr_v.at[b], out_hbm.at[pl.ds((ct0 + i) * 64, 64)],
                osem.at[b]).wait()

        def transpose_block(b):
            def c_body(c, carry):
                vloc = c * 16 + lanes
                prow = lax.shift_right_logical(vloc, 1)
                colb = (vloc & 1) * D

                def d_body(dd, carry2):
                    for u in range(4):
                        d = dd * 4 + u
                        vals = in_v[b, d, pl.ds(c * 16, 16)]
                        plsc.store_scatter(tr_v.at[b], [prow, colb + d], vals)
                    return carry2

                return lax.fori_loop(0, D // 4, d_body, carry)

            lax.fori_loop(0, 8, c_body, 0)

        fire_in(0, 0)

        def steady(i, carry):
            b = lax.rem(i, 2)
            bn = lax.rem(i + 1, 2)

            @pl.when(i + 1 < nct)
            def _():
                fire_in(i + 1, bn)

            wait_in(i, b)

            @pl.when(i >= 2)
            def _():
                wait_out(i - 2, b)

            transpose_block(b)
            fire_out(i, b)
            return carry

        lax.fori_loop(0, nct, steady, 0)
        wait_out(nct - 2, lax.rem(nct - 2, 2))
        wait_out(nct - 1, lax.rem(nct - 1, 2))

        @pl.when(wid == 0)
        def _():
            pltpu.sync_copy(tailp_hbm, tail_v)
            pltpu.sync_copy(tail_v, out_hbm.at[pl.ds(VMAIN // 2, 32)])

    return body


def _phase_b(n_rows):
    per_w = n_rows // NW
    n_chunks = per_w // CHUNK
    mesh = plsc.VectorSubcoreMesh(core_axis_name="c", subcore_axis_name="s")

    @functools.partial(
        pl.kernel,
        mesh=mesh,
        compiler_params=pltpu.CompilerParams(use_tc_tiling_on_sc=False),
        out_type=jax.ShapeDtypeStruct((n_rows, 128), jnp.float32),
        scratch_types=[
            pltpu.VMEM((n_chunks, CHUNK), jnp.int32),
            pltpu.VMEM((PERIOD, D), jnp.float32),
            pltpu.VMEM((4, CHUNK, D), jnp.float32),
            pltpu.SemaphoreType.DMA((4,)),
            pltpu.SemaphoreType.DMA((4,)),
        ],
    )
    def body(tab_hbm, idx_hbm, pe_hbm, out_hbm, idx_v, pe_v, rows_v, gsem, ssem):
        wid = lax.axis_index("s") * NC + lax.axis_index("c")
        rbase = wid * per_w

        pltpu.sync_copy(idx_hbm.at[wid], idx_v)
        pltpu.sync_copy(pe_hbm, pe_v)

        def fire_gather(j, b):
            pltpu.async_copy(tab_hbm.at[idx_v.at[j]], rows_v.at[b], gsem.at[b])

        def wait_gather(j, b):
            pltpu.make_async_copy(
                tab_hbm.at[idx_v.at[j]], rows_v.at[b], gsem.at[b]).wait()

        def fire_store(j, b):
            pltpu.async_copy(rows_v.at[b],
                             out_hbm.at[pl.ds(rbase + j * CHUNK, CHUNK),
                                        pl.ds(0, D)],
                             ssem.at[b])

        def wait_store(j, b):
            pltpu.make_async_copy(
                rows_v.at[b],
                out_hbm.at[pl.ds(rbase + j * CHUNK, CHUNK), pl.ds(0, D)],
                ssem.at[b]).wait()

        def add_pe(b, parity):
            pbase = parity * CHUNK

            def row_body(r, carry):
                for c in range(D // 16):
                    vec = pe_v[pbase + r, pl.ds(c * 16, 16)]
                    plsc.addupdate(rows_v.at[b, r, pl.ds(c * 16, 16)], vec)
                return carry

            lax.fori_loop(0, CHUNK, row_body, 0, unroll=4)

        fire_gather(0, 0)
        fire_gather(1, 1)

        wait_gather(0, 0)
        add_pe(0, 0)
        fire_store(0, 0)
        fire_gather(2, 2)

        wait_gather(1, 1)
        add_pe(1, 1)
        fire_store(1, 1)
        fire_gather(3, 3)

        def steady(jj, carry):
            j0 = 2 + jj * 4
            for b_off in range(4):
                j = j0 + b_off
                b = (2 + b_off) % 4
                parity = b_off % 2
                wait_gather(j, b)
                add_pe(b, parity)
                fire_store(j, b)
                wait_store(j - 2, (b + 2) % 4)
                fire_gather(j + 2, (b + 2) % 4)
            return carry

        lax.fori_loop(0, (n_chunks - 4) // 4, steady, 0)

        jt = n_chunks - 2
        wait_gather(jt, jt % 4)
        add_pe(jt % 4, jt % 2)
        fire_store(jt, jt % 4)
        wait_store(jt - 2, (jt - 2) % 4)

        jt = n_chunks - 1
        wait_gather(jt, jt % 4)
        add_pe(jt % 4, jt % 2)
        fire_store(jt, jt % 4)
        wait_store(jt - 2, (jt - 2) % 4)

        wait_store(n_chunks - 2, (n_chunks - 2) % 4)
        wait_store(n_chunks - 1, (n_chunks - 1) % 4)

    return body


def kernel(sequence, token_table):
    batch, seq_len = sequence.shape
    vocab, d_model = token_table.shape
    n_rows = batch * seq_len
    pe = _pos_table(seq_len, d_model)
    idx = sequence.reshape(NW, n_rows // NW // CHUNK, CHUNK).astype(jnp.int32)

    tabT = token_table.T
    tailp = token_table[VMAIN:].reshape(32, 128)
    tab_pairs = _phase_a()(tabT, tailp)
    tab_flat = tab_pairs.reshape(vocab, d_model)
    out = _phase_b(n_rows)(tab_flat, idx, pe)
    return out[:, :d_model].reshape(batch, seq_len, d_model)
